# baseline (speedup 1.0000x reference)
import functools

import jax
import jax.numpy as jnp
from jax import lax
from jax.experimental import pallas as pl
from jax.experimental.pallas import tpu as pltpu
from jax.experimental.pallas import tpu_sc as plsc

N = 10000
E = 320000
C = 128
H = 4
HC = H * C

NC = 2
NS = 16
NW = NC * NS
EPT = 10048
B = 64
NBLK = EPT // B
E_PAD = NW * EPT
ND = H * EPT
NP_OUT = 10240

_mesh = plsc.VectorSubcoreMesh(
    core_axis_name="c", subcore_axis_name="s", num_cores=NC, num_subcores=NS)


def _mm_body(x_ref, wl_ref, wr_ref, bl_ref, br_ref, xl_ref, xr_ref):
    xv = x_ref[...]
    xl_ref[...] = (jnp.dot(xv, wl_ref[...], preferred_element_type=jnp.float32)
                   + bl_ref[...]).astype(jnp.bfloat16)
    xr_ref[...] = (jnp.dot(xv, wr_ref[...], preferred_element_type=jnp.float32)
                   + br_ref[...]).astype(jnp.bfloat16)


def _matmuls(x, W_l, W_r, b_l, b_r):
    blk = 1000
    grid = (N // blk,)
    return pl.pallas_call(
        _mm_body,
        grid=grid,
        in_specs=[
            pl.BlockSpec((blk, C), lambda i: (i, 0)),
            pl.BlockSpec((C, HC), lambda i: (0, 0)),
            pl.BlockSpec((C, HC), lambda i: (0, 0)),
            pl.BlockSpec((1, HC), lambda i: (0, 0)),
            pl.BlockSpec((1, HC), lambda i: (0, 0)),
        ],
        out_specs=[
            pl.BlockSpec((blk, HC), lambda i: (i, 0)),
            pl.BlockSpec((blk, HC), lambda i: (i, 0)),
        ],
        out_shape=[
            jax.ShapeDtypeStruct((N, HC), jnp.bfloat16),
            jax.ShapeDtypeStruct((N, HC), jnp.bfloat16),
        ],
    )(x, W_l, W_r, b_l.reshape(1, HC), b_r.reshape(1, HC))


def _p1_body(xl_hbm, xr_hbm, pk_hbm, att_hbm,
             ex_hbm, den_hbm,
             idxb, xlbuf, xrbuf, lscr, exblk, att_v, den_v,
             sem1, sem2, semi, seme):
    c = lax.axis_index("c")
    s = lax.axis_index("s")
    wid = s * NC + c

    pltpu.sync_copy(att_hbm, att_v)
    attv = [plsc.bitcast(att_v[pl.ds(16 * p, 16)], jnp.bfloat16)
            for p in range(16)]
    zeros16 = jnp.zeros((16,), jnp.float32)

    @pl.loop(0, ND // 16)
    def _zero(i):
        den_v[pl.ds(i * 16, 16)] = zeros16

    gblk0 = wid * NBLK
    iota = lax.iota(jnp.int32, 16)
    onehot0 = (iota == 0).astype(jnp.float32)

    def start_gathers(bi, bg):
        pltpu.async_copy(xl_hbm.at[idxb.at[bi, 0]], xlbuf.at[bg], sem1)
        pltpu.async_copy(xr_hbm.at[idxb.at[bi, 1]], xrbuf.at[bg], sem2)

    def wait_gathers(bi, bg):
        pltpu.make_async_copy(xl_hbm.at[idxb.at[bi, 0]], xlbuf.at[bg], sem1).wait()
        pltpu.make_async_copy(xr_hbm.at[idxb.at[bi, 1]], xrbuf.at[bg], sem2).wait()

    pltpu.sync_copy(pk_hbm.at[gblk0], idxb.at[0])
    start_gathers(0, 0)
    pltpu.async_copy(pk_hbm.at[gblk0 + 1], idxb.at[1], semi)

    @pl.loop(0, NBLK)
    def _blk(blk):
        par = blk & 1
        parn = 1 - par
        i_cur = lax.rem(blk, 3)
        i_next = lax.rem(blk + 1, 3)
        i_pref = lax.rem(blk + 2, 3)
        wait_gathers(i_cur, par)

        @pl.when(blk + 1 < NBLK)
        def _next():
            pltpu.make_async_copy(
                pk_hbm.at[gblk0 + blk + 1], idxb.at[i_next], semi).wait()
            start_gathers(i_next, parn)

        @pl.when(blk + 2 < NBLK)
        def _pref():
            pltpu.async_copy(pk_hbm.at[gblk0 + blk + 2], idxb.at[i_pref], semi)

        @pl.loop(0, B, unroll=4)
        def _edge(e):
            for h in range(H):
                acc = zeros16
                for p4 in range(4):
                    p = h * 4 + p4
                    lw = plsc.bitcast(xlbuf[par, e, pl.ds(16 * p, 16)],
                                      jnp.bfloat16)
                    rw = plsc.bitcast(xrbuf[par, e, pl.ds(16 * p, 16)],
                                      jnp.bfloat16)
                    sv = lw + rw
                    lv = jnp.maximum(sv, jnp.bfloat16(0.2) * sv)
                    pr = lv * attv[p]
                    lo, hi = plsc.unpack(pr, format=plsc.PackFormat.INTERLEAVED)
                    acc = acc + lo + hi
                lscr[pl.ds(e * 64 + h * 16, 16)] = acc

        @pl.when(blk >= 2)
        def _draine():
            pltpu.make_async_copy(
                exblk.at[pl.ds(0, B * 4)], ex_hbm.at[pl.ds(0, B * 4)], seme).wait()

        for g in range(B // 16):
            dstv = idxb[i_cur, 1, pl.ds(g * 16, 16)]
            for h in range(H):
                tot = zeros16
                for j in range(16):
                    tot = tot + plsc.load_gather(
                        lscr, [iota * 64 + (g * 1024 + h * 16 + j)])
                exv = jnp.exp(tot)
                plsc.store_scatter(
                    exblk, [par * B * 4 + iota * 4 + (g * 64 + h)], exv)
                for j in range(16):
                    idx = dstv[j] * 4 + h
                    plsc.addupdate(den_v.at[pl.ds(idx, 16)], exv[j] * onehot0)

        base = (gblk0 + blk) * B
        pltpu.async_copy(exblk.at[pl.ds(par * B * 4, B * 4)],
                         ex_hbm.at[pl.ds(base * 4, B * 4)], seme)

    pltpu.make_async_copy(
        exblk.at[pl.ds(0, B * 4)], ex_hbm.at[pl.ds(0, B * 4)], seme).wait()
    pltpu.make_async_copy(
        exblk.at[pl.ds(0, B * 4)], ex_hbm.at[pl.ds(0, B * 4)], seme).wait()
    pltpu.sync_copy(den_v, den_hbm.at[wid])


@functools.partial(
    pl.kernel,
    out_type=(
        jax.ShapeDtypeStruct((E_PAD * 4,), jnp.float32),
        jax.ShapeDtypeStruct((NW, ND), jnp.float32),
    ),
    mesh=_mesh,
    scratch_types=[
        pltpu.VMEM((3, 2, B), jnp.int32),
        pltpu.VMEM((2, B, HC // 2), jnp.int32),
        pltpu.VMEM((2, B, HC // 2), jnp.int32),
        pltpu.VMEM((B * 64,), jnp.float32),
        pltpu.VMEM((2 * B * 4,), jnp.float32),
        pltpu.VMEM((HC // 2,), jnp.int32),
        pltpu.VMEM((ND,), jnp.float32),
        pltpu.SemaphoreType.DMA,
        pltpu.SemaphoreType.DMA,
        pltpu.SemaphoreType.DMA,
        pltpu.SemaphoreType.DMA,
    ],
    compiler_params=pltpu.CompilerParams(needs_layout_passes=False),
)
def _phase1(*refs):
    _p1_body(*refs)


def _mid_body(den_ref, rden_ref):
    d = jnp.sum(den_ref[...], axis=0) + 1e-16
    rden_ref[...] = 1.0 / d


def _mid(den):
    return pl.pallas_call(
        _mid_body,
        out_shape=jax.ShapeDtypeStruct((ND // 128, 128), jnp.float32),
    )(den.reshape(NW, ND // 128, 128))


def _p2_body(ex_hbm, dst_hbm, rden_hbm, al_hbm, didx, exblk, rden_v):
    c = lax.axis_index("c")
    s = lax.axis_index("s")
    wid = s * NC + c
    base = wid * EPT

    pltpu.sync_copy(rden_hbm, rden_v)
    pltpu.sync_copy(dst_hbm.at[pl.ds(base, EPT)], didx)
    pltpu.sync_copy(ex_hbm.at[pl.ds(base * 4, EPT * 4)], exblk)
    iota = lax.iota(jnp.int32, 16)

    @pl.loop(0, EPT // 16)
    def _grp(g):
        dstv = didx[pl.ds(g * 16, 16)]
        dst4 = dstv * 4
        for h in range(H):
            exv = plsc.load_gather(exblk, [iota * 4 + (g * 64 + h)])
            rd = plsc.load_gather(rden_v, [dst4 + h])
            plsc.store_scatter(exblk, [iota * 4 + (g * 64 + h)], exv * rd)

    pltpu.sync_copy(exblk, al_hbm.at[pl.ds(base * 4, EPT * 4)])


@functools.partial(
    pl.kernel,
    out_type=jax.ShapeDtypeStruct((E_PAD * 4,), jnp.float32),
    mesh=_mesh,
    scratch_types=[
        pltpu.VMEM((EPT,), jnp.int32),
        pltpu.VMEM((EPT * 4,), jnp.float32),
        pltpu.VMEM((ND,), jnp.float32),
    ],
    compiler_params=pltpu.CompilerParams(needs_layout_passes=False),
)
def _phase2(*refs):
    _p2_body(*refs)


def _p3_body(xl_hbm, pk_hbm, al_hbm,
             outp_hbm,
             idxb, xlbuf, albuf, vbuf, out_sh, sem1, sema, semi):
    c = lax.axis_index("c")
    s = lax.axis_index("s")
    wid = s * NC + c

    zeros16 = jnp.zeros((16,), jnp.float32)

    @pl.loop(0, B)
    def _zv(e):
        for j8 in range(8):
            vbuf[e, pl.ds(16 * j8, 16)] = zeros16

    @pl.loop(0, NP_OUT // (NS * B))
    def _zo(t):
        pltpu.sync_copy(vbuf, out_sh.at[pl.ds(s * (NP_OUT // NS) + t * B, B)])

    plsc.subcore_barrier()

    gblk0 = wid * NBLK

    def start_fetch(bi, bg, gblk):
        pltpu.async_copy(xl_hbm.at[idxb.at[bi, 0]], xlbuf.at[bg], sem1)
        pltpu.async_copy(al_hbm.at[pl.ds(gblk * (B * 4), B * 4)],
                         albuf.at[bg, pl.ds(0, B * 4)], sema)

    def wait_fetch(bi, bg):
        pltpu.make_async_copy(xl_hbm.at[idxb.at[bi, 0]], xlbuf.at[bg], sem1).wait()
        pltpu.make_async_copy(al_hbm.at[pl.ds(0, B * 4)],
                              albuf.at[bg, pl.ds(0, B * 4)], sema).wait()

    pltpu.sync_copy(pk_hbm.at[gblk0], idxb.at[0])
    start_fetch(0, 0, gblk0)
    pltpu.async_copy(pk_hbm.at[gblk0 + 1], idxb.at[1], semi)

    @pl.loop(0, NBLK)
    def _blk(blk):
        par = blk & 1
        parn = 1 - par
        i_cur = lax.rem(blk, 3)
        i_next = lax.rem(blk + 1, 3)
        i_pref = lax.rem(blk + 2, 3)
        wait_fetch(i_cur, par)

        @pl.when(blk + 1 < NBLK)
        def _next():
            pltpu.make_async_copy(
                pk_hbm.at[gblk0 + blk + 1], idxb.at[i_next], semi).wait()
            start_fetch(i_next, parn, gblk0 + blk + 1)

        @pl.when(blk + 2 < NBLK)
        def _pref():
            pltpu.async_copy(pk_hbm.at[gblk0 + blk + 2], idxb.at[i_pref], semi)

        @pl.loop(0, B, unroll=4)
        def _edge(e):
            av = albuf[par, pl.ds(e * 4, 16)]
            aa = [av[0], av[1], av[2], av[3]]
            for p in range(4):
                acclo = jnp.zeros((16,), jnp.float32)
                acchi = jnp.zeros((16,), jnp.float32)
                for h in range(H):
                    xh = plsc.bitcast(
                        xlbuf[par, e, pl.ds(h * 64 + p * 16, 16)], jnp.bfloat16)
                    lo, hi = plsc.unpack(xh, format=plsc.PackFormat.INTERLEAVED)
                    acclo = acclo + aa[h] * lo
                    acchi = acchi + aa[h] * hi
                vbuf[e, pl.ds(p * 32, 16)] = acclo
                vbuf[e, pl.ds(p * 32 + 16, 16)] = acchi

        pltpu.sync_copy(vbuf, out_sh.at[idxb.at[i_cur, 1]], add=True)

    plsc.subcore_barrier()
    rows = NP_OUT // NS
    pltpu.sync_copy(out_sh.at[pl.ds(s * rows, rows)],
                    outp_hbm.at[c, pl.ds(s * rows, rows)])


@functools.partial(
    pl.kernel,
    out_type=jax.ShapeDtypeStruct((NC, NP_OUT, C), jnp.float32),
    mesh=_mesh,
    scratch_types=[
        pltpu.VMEM((3, 2, B), jnp.int32),
        pltpu.VMEM((2, B, HC // 2), jnp.int32),
        pltpu.VMEM((2, B * 4 + 16), jnp.float32),
        pltpu.VMEM((B, C), jnp.float32),
        pltpu.VMEM_SHARED((NP_OUT, C), jnp.float32),
        pltpu.SemaphoreType.DMA,
        pltpu.SemaphoreType.DMA,
        pltpu.SemaphoreType.DMA,
    ],
    compiler_params=pltpu.CompilerParams(needs_layout_passes=False),
)
def _phase3(*refs):
    _p3_body(*refs)


def _fin_body(p_ref, x_ref, b_ref, gw_ref, gb_ref, gms_ref, o_ref):
    p = p_ref[0, :N, :] + p_ref[1, :N, :]
    out0 = p * (1.0 / H) + b_ref[...]
    mean = jnp.mean(out0, axis=0, keepdims=True)
    outc = out0 - gms_ref[...] * mean
    var = jnp.mean(outc * outc, axis=0, keepdims=True)
    y = outc * lax.rsqrt(var + 1e-5) * gw_ref[...] + gb_ref[...]
    y = jnp.where(y > 0, y, jnp.exp(y) - 1.0)
    o_ref[...] = y + x_ref[...]


def _final(outp, x, bias, gn_weight, gn_bias, gn_mean_scale):
    return pl.pallas_call(
        _fin_body,
        out_shape=jax.ShapeDtypeStruct((N, C), jnp.float32),
    )(outp, x, bias.reshape(1, C), gn_weight.reshape(1, C),
      gn_bias.reshape(1, C), gn_mean_scale.reshape(1, C))


def kernel(x, edge_index, W_l, b_l, W_r, b_r, att, bias, gn_weight,
           gn_bias, gn_mean_scale):
    ei = edge_index.astype(jnp.int32)
    pad = E_PAD - E
    src = jnp.concatenate([ei[0], jnp.zeros((pad,), jnp.int32)])
    dst = jnp.concatenate([ei[1], jnp.full((pad,), N, jnp.int32)])
    pk = jnp.stack([src.reshape(-1, B), dst.reshape(-1, B)], axis=1)

    perm = jnp.asarray(
        [32 * g + off for g in range(HC // 32) for k in range(16)
         for off in (k, 16 + k)], dtype=jnp.int32)
    Wlp = jnp.take(W_l, perm, axis=1)
    Wrp = jnp.take(W_r, perm, axis=1)
    blp = jnp.take(b_l, perm)
    brp = jnp.take(b_r, perm)
    attp = lax.bitcast_convert_type(
        jnp.take(att.reshape(HC), perm).astype(jnp.bfloat16).reshape(HC // 2, 2),
        jnp.int32)

    xlb, xrb = _matmuls(x, Wlp, Wrp, blp, brp)
    xl32 = lax.bitcast_convert_type(xlb.reshape(N, HC // 2, 2), jnp.int32)
    xr32 = lax.bitcast_convert_type(xrb.reshape(N, HC // 2, 2), jnp.int32)
    ex, den = _phase1(xl32, xr32, pk, attp)
    rden = _mid(den)
    al = _phase2(ex, dst, rden.reshape(ND))
    outp = _phase3(xl32, pk, al)
    return _final(outp, x, bias, gn_weight, gn_bias, gn_mean_scale)


# speedup vs baseline: 1.2037x; 1.2037x over previous
import functools

import jax
import jax.numpy as jnp
from jax import lax
from jax.experimental import pallas as pl
from jax.experimental.pallas import tpu as pltpu
from jax.experimental.pallas import tpu_sc as plsc

N = 10000
E = 320000
C = 128
H = 4
HC = H * C

NC = 2
NS = 16
NW = NC * NS
EPT = 10048
B = 64
NBLK = EPT // B
E_PAD = NW * EPT
ND = H * EPT
NP_OUT = 10240

_mesh = plsc.VectorSubcoreMesh(
    core_axis_name="c", subcore_axis_name="s", num_cores=NC, num_subcores=NS)


def _pack_rows(y):
    bits = lax.bitcast_convert_type(y, jnp.int32)
    lo = lax.shift_right_logical(bits[:, :HC // 2] + 0x8000, 16)
    hi = jnp.bitwise_and(bits[:, HC // 2:] + 0x8000, jnp.int32(-65536))
    return jnp.bitwise_or(lo, hi)


def _mm_body(x_ref, wl_ref, wr_ref, bl_ref, br_ref, xl_ref, xr_ref):
    xv = x_ref[...]
    xl_ref[...] = _pack_rows(
        jnp.dot(xv, wl_ref[...], preferred_element_type=jnp.float32)
        + bl_ref[...])
    xr_ref[...] = _pack_rows(
        jnp.dot(xv, wr_ref[...], preferred_element_type=jnp.float32)
        + br_ref[...])


def _matmuls(x, W_l, W_r, b_l, b_r):
    blk = 1000
    grid = (N // blk,)
    return pl.pallas_call(
        _mm_body,
        grid=grid,
        in_specs=[
            pl.BlockSpec((blk, C), lambda i: (i, 0)),
            pl.BlockSpec((C, HC), lambda i: (0, 0)),
            pl.BlockSpec((C, HC), lambda i: (0, 0)),
            pl.BlockSpec((1, HC), lambda i: (0, 0)),
            pl.BlockSpec((1, HC), lambda i: (0, 0)),
        ],
        out_specs=[
            pl.BlockSpec((blk, HC // 2), lambda i: (i, 0)),
            pl.BlockSpec((blk, HC // 2), lambda i: (i, 0)),
        ],
        out_shape=[
            jax.ShapeDtypeStruct((N, HC // 2), jnp.int32),
            jax.ShapeDtypeStruct((N, HC // 2), jnp.int32),
        ],
    )(x, W_l, W_r, b_l.reshape(1, HC), b_r.reshape(1, HC))


def _p1_body(xl_hbm, xr_hbm, pk_hbm, att_hbm,
             ex_hbm, den_hbm,
             idxb, xlbuf, xrbuf, lscr, exblk, att_v, den_v,
             sem1, sem2, semi, seme):
    c = lax.axis_index("c")
    s = lax.axis_index("s")
    wid = s * NC + c

    pltpu.sync_copy(att_hbm, att_v)
    attv = [plsc.bitcast(att_v[pl.ds(16 * p, 16)], jnp.bfloat16)
            for p in range(16)]
    zeros16 = jnp.zeros((16,), jnp.float32)

    @pl.loop(0, ND // 16)
    def _zero(i):
        den_v[pl.ds(i * 16, 16)] = zeros16

    gblk0 = wid * NBLK
    iota = lax.iota(jnp.int32, 16)
    onehot0 = (iota == 0).astype(jnp.float32)

    def start_gathers(bi, bg):
        pltpu.async_copy(xl_hbm.at[idxb.at[bi, 0]], xlbuf.at[bg], sem1)
        pltpu.async_copy(xr_hbm.at[idxb.at[bi, 1]], xrbuf.at[bg], sem2)

    def wait_gathers(bi, bg):
        pltpu.make_async_copy(xl_hbm.at[idxb.at[bi, 0]], xlbuf.at[bg], sem1).wait()
        pltpu.make_async_copy(xr_hbm.at[idxb.at[bi, 1]], xrbuf.at[bg], sem2).wait()

    pltpu.sync_copy(pk_hbm.at[gblk0], idxb.at[0])
    start_gathers(0, 0)
    pltpu.async_copy(pk_hbm.at[gblk0 + 1], idxb.at[1], semi)

    @pl.loop(0, NBLK)
    def _blk(blk):
        par = blk & 1
        parn = 1 - par
        i_cur = lax.rem(blk, 3)
        i_next = lax.rem(blk + 1, 3)
        i_pref = lax.rem(blk + 2, 3)
        wait_gathers(i_cur, par)

        @pl.when(blk + 1 < NBLK)
        def _next():
            pltpu.make_async_copy(
                pk_hbm.at[gblk0 + blk + 1], idxb.at[i_next], semi).wait()
            start_gathers(i_next, parn)

        @pl.when(blk + 2 < NBLK)
        def _pref():
            pltpu.async_copy(pk_hbm.at[gblk0 + blk + 2], idxb.at[i_pref], semi)

        @pl.loop(0, B, unroll=4)
        def _edge(e):
            for h in range(H):
                acc = zeros16
                for p4 in range(4):
                    p = h * 4 + p4
                    lw = plsc.bitcast(xlbuf[par, e, pl.ds(16 * p, 16)],
                                      jnp.bfloat16)
                    rw = plsc.bitcast(xrbuf[par, e, pl.ds(16 * p, 16)],
                                      jnp.bfloat16)
                    sv = lw + rw
                    lv = jnp.maximum(sv, jnp.bfloat16(0.2) * sv)
                    pr = lv * attv[p]
                    lo, hi = plsc.unpack(pr, format=plsc.PackFormat.INTERLEAVED)
                    acc = acc + lo + hi
                lscr[pl.ds(e * 64 + h * 16, 16)] = acc

        @pl.when(blk >= 2)
        def _draine():
            pltpu.make_async_copy(
                exblk.at[pl.ds(0, B * 4)], ex_hbm.at[pl.ds(0, B * 4)], seme).wait()

        for g in range(B // 16):
            dstv = idxb[i_cur, 1, pl.ds(g * 16, 16)]
            for h in range(H):
                tot = zeros16
                for j in range(16):
                    tot = tot + plsc.load_gather(
                        lscr, [iota * 64 + (g * 1024 + h * 16 + j)])
                exv = jnp.exp(tot)
                plsc.store_scatter(
                    exblk, [par * B * 4 + iota * 4 + (g * 64 + h)], exv)
                for j in range(16):
                    idx = dstv[j] * 4 + h
                    plsc.addupdate(den_v.at[pl.ds(idx, 16)], exv[j] * onehot0)

        base = (gblk0 + blk) * B
        pltpu.async_copy(exblk.at[pl.ds(par * B * 4, B * 4)],
                         ex_hbm.at[pl.ds(base * 4, B * 4)], seme)

    pltpu.make_async_copy(
        exblk.at[pl.ds(0, B * 4)], ex_hbm.at[pl.ds(0, B * 4)], seme).wait()
    pltpu.make_async_copy(
        exblk.at[pl.ds(0, B * 4)], ex_hbm.at[pl.ds(0, B * 4)], seme).wait()
    pltpu.sync_copy(den_v, den_hbm.at[wid])


@functools.partial(
    pl.kernel,
    out_type=(
        jax.ShapeDtypeStruct((E_PAD * 4,), jnp.float32),
        jax.ShapeDtypeStruct((NW, ND), jnp.float32),
    ),
    mesh=_mesh,
    scratch_types=[
        pltpu.VMEM((3, 2, B), jnp.int32),
        pltpu.VMEM((2, B, HC // 2), jnp.int32),
        pltpu.VMEM((2, B, HC // 2), jnp.int32),
        pltpu.VMEM((B * 64,), jnp.float32),
        pltpu.VMEM((2 * B * 4,), jnp.float32),
        pltpu.VMEM((HC // 2,), jnp.int32),
        pltpu.VMEM((ND,), jnp.float32),
        pltpu.SemaphoreType.DMA,
        pltpu.SemaphoreType.DMA,
        pltpu.SemaphoreType.DMA,
        pltpu.SemaphoreType.DMA,
    ],
    compiler_params=pltpu.CompilerParams(needs_layout_passes=False),
)
def _phase1(*refs):
    _p1_body(*refs)


def _mid_body(den_ref, rden_ref):
    d = jnp.sum(den_ref[...], axis=0) + 1e-16
    rden_ref[...] = 1.0 / d


def _mid(den):
    return pl.pallas_call(
        _mid_body,
        out_shape=jax.ShapeDtypeStruct((ND // 128, 128), jnp.float32),
    )(den.reshape(NW, ND // 128, 128))


def _p2_body(ex_hbm, dst_hbm, rden_hbm, al_hbm, didx, exblk, rden_v):
    c = lax.axis_index("c")
    s = lax.axis_index("s")
    wid = s * NC + c
    base = wid * EPT

    pltpu.sync_copy(rden_hbm, rden_v)
    pltpu.sync_copy(dst_hbm.at[pl.ds(base, EPT)], didx)
    pltpu.sync_copy(ex_hbm.at[pl.ds(base * 4, EPT * 4)], exblk)
    iota = lax.iota(jnp.int32, 16)

    @pl.loop(0, EPT // 16)
    def _grp(g):
        dstv = didx[pl.ds(g * 16, 16)]
        dst4 = dstv * 4
        for h in range(H):
            exv = plsc.load_gather(exblk, [iota * 4 + (g * 64 + h)])
            rd = plsc.load_gather(rden_v, [dst4 + h])
            plsc.store_scatter(exblk, [iota * 4 + (g * 64 + h)], exv * rd)

    pltpu.sync_copy(exblk, al_hbm.at[pl.ds(base * 4, EPT * 4)])


@functools.partial(
    pl.kernel,
    out_type=jax.ShapeDtypeStruct((E_PAD * 4,), jnp.float32),
    mesh=_mesh,
    scratch_types=[
        pltpu.VMEM((EPT,), jnp.int32),
        pltpu.VMEM((EPT * 4,), jnp.float32),
        pltpu.VMEM((ND,), jnp.float32),
    ],
    compiler_params=pltpu.CompilerParams(needs_layout_passes=False),
)
def _phase2(*refs):
    _p2_body(*refs)


def _p3_body(xl_hbm, pk_hbm, al_hbm,
             outp_hbm,
             idxb, xlbuf, albuf, vbuf, out_sh, sem1, sema, semi):
    c = lax.axis_index("c")
    s = lax.axis_index("s")
    wid = s * NC + c

    zeros16 = jnp.zeros((16,), jnp.float32)

    @pl.loop(0, B)
    def _zv(e):
        for j8 in range(8):
            vbuf[e, pl.ds(16 * j8, 16)] = zeros16

    @pl.loop(0, NP_OUT // (NS * B))
    def _zo(t):
        pltpu.sync_copy(vbuf, out_sh.at[pl.ds(s * (NP_OUT // NS) + t * B, B)])

    plsc.subcore_barrier()

    gblk0 = wid * NBLK

    def start_fetch(bi, bg, gblk):
        pltpu.async_copy(xl_hbm.at[idxb.at[bi, 0]], xlbuf.at[bg], sem1)
        pltpu.async_copy(al_hbm.at[pl.ds(gblk * (B * 4), B * 4)],
                         albuf.at[bg, pl.ds(0, B * 4)], sema)

    def wait_fetch(bi, bg):
        pltpu.make_async_copy(xl_hbm.at[idxb.at[bi, 0]], xlbuf.at[bg], sem1).wait()
        pltpu.make_async_copy(al_hbm.at[pl.ds(0, B * 4)],
                              albuf.at[bg, pl.ds(0, B * 4)], sema).wait()

    pltpu.sync_copy(pk_hbm.at[gblk0], idxb.at[0])
    start_fetch(0, 0, gblk0)
    pltpu.async_copy(pk_hbm.at[gblk0 + 1], idxb.at[1], semi)

    @pl.loop(0, NBLK)
    def _blk(blk):
        par = blk & 1
        parn = 1 - par
        i_cur = lax.rem(blk, 3)
        i_next = lax.rem(blk + 1, 3)
        i_pref = lax.rem(blk + 2, 3)
        wait_fetch(i_cur, par)

        @pl.when(blk + 1 < NBLK)
        def _next():
            pltpu.make_async_copy(
                pk_hbm.at[gblk0 + blk + 1], idxb.at[i_next], semi).wait()
            start_fetch(i_next, parn, gblk0 + blk + 1)

        @pl.when(blk + 2 < NBLK)
        def _pref():
            pltpu.async_copy(pk_hbm.at[gblk0 + blk + 2], idxb.at[i_pref], semi)

        @pl.loop(0, B, unroll=4)
        def _edge(e):
            av = albuf[par, pl.ds(e * 4, 16)]
            aa = [av[0], av[1], av[2], av[3]]
            for p in range(4):
                acclo = jnp.zeros((16,), jnp.float32)
                acchi = jnp.zeros((16,), jnp.float32)
                for h in range(H):
                    xh = plsc.bitcast(
                        xlbuf[par, e, pl.ds(h * 64 + p * 16, 16)], jnp.bfloat16)
                    lo, hi = plsc.unpack(xh, format=plsc.PackFormat.INTERLEAVED)
                    acclo = acclo + aa[h] * lo
                    acchi = acchi + aa[h] * hi
                vbuf[e, pl.ds(p * 32, 16)] = acclo
                vbuf[e, pl.ds(p * 32 + 16, 16)] = acchi

        pltpu.sync_copy(vbuf, out_sh.at[idxb.at[i_cur, 1]], add=True)

    plsc.subcore_barrier()
    rows = NP_OUT // NS
    pltpu.sync_copy(out_sh.at[pl.ds(s * rows, rows)],
                    outp_hbm.at[c, pl.ds(s * rows, rows)])


@functools.partial(
    pl.kernel,
    out_type=jax.ShapeDtypeStruct((NC, NP_OUT, C), jnp.float32),
    mesh=_mesh,
    scratch_types=[
        pltpu.VMEM((3, 2, B), jnp.int32),
        pltpu.VMEM((2, B, HC // 2), jnp.int32),
        pltpu.VMEM((2, B * 4 + 16), jnp.float32),
        pltpu.VMEM((B, C), jnp.float32),
        pltpu.VMEM_SHARED((NP_OUT, C), jnp.float32),
        pltpu.SemaphoreType.DMA,
        pltpu.SemaphoreType.DMA,
        pltpu.SemaphoreType.DMA,
    ],
    compiler_params=pltpu.CompilerParams(needs_layout_passes=False),
)
def _phase3(*refs):
    _p3_body(*refs)


def _fin_body(p_ref, x_ref, b_ref, gw_ref, gb_ref, gms_ref, o_ref):
    p = p_ref[0, :N, :] + p_ref[1, :N, :]
    out0 = p * (1.0 / H) + b_ref[...]
    mean = jnp.mean(out0, axis=0, keepdims=True)
    outc = out0 - gms_ref[...] * mean
    var = jnp.mean(outc * outc, axis=0, keepdims=True)
    y = outc * lax.rsqrt(var + 1e-5) * gw_ref[...] + gb_ref[...]
    y = jnp.where(y > 0, y, jnp.exp(y) - 1.0)
    o_ref[...] = y + x_ref[...]


def _final(outp, x, bias, gn_weight, gn_bias, gn_mean_scale):
    return pl.pallas_call(
        _fin_body,
        out_shape=jax.ShapeDtypeStruct((N, C), jnp.float32),
    )(outp, x, bias.reshape(1, C), gn_weight.reshape(1, C),
      gn_bias.reshape(1, C), gn_mean_scale.reshape(1, C))


def kernel(x, edge_index, W_l, b_l, W_r, b_r, att, bias, gn_weight,
           gn_bias, gn_mean_scale):
    ei = edge_index.astype(jnp.int32)
    pad = E_PAD - E
    src = jnp.concatenate([ei[0], jnp.zeros((pad,), jnp.int32)])
    dst = jnp.concatenate([ei[1], jnp.full((pad,), N, jnp.int32)])
    pk = jnp.stack([src.reshape(-1, B), dst.reshape(-1, B)], axis=1)

    perm = jnp.asarray(
        [32 * (w // 16) + (w % 16) for w in range(HC // 2)]
        + [32 * (w // 16) + 16 + (w % 16) for w in range(HC // 2)],
        dtype=jnp.int32)
    perm_att = jnp.asarray(
        [32 * g + off for g in range(HC // 32) for k in range(16)
         for off in (k, 16 + k)], dtype=jnp.int32)
    Wlp = jnp.take(W_l, perm, axis=1)
    Wrp = jnp.take(W_r, perm, axis=1)
    blp = jnp.take(b_l, perm)
    brp = jnp.take(b_r, perm)
    attp = lax.bitcast_convert_type(
        jnp.take(att.reshape(HC), perm_att).astype(jnp.bfloat16)
        .reshape(HC // 2, 2), jnp.int32)

    xl32, xr32 = _matmuls(x, Wlp, Wrp, blp, brp)
    ex, den = _phase1(xl32, xr32, pk, attp)
    rden = _mid(den)
    al = _phase2(ex, dst, rden.reshape(ND))
    outp = _phase3(xl32, pk, al)
    return _final(outp, x, bias, gn_weight, gn_bias, gn_mean_scale)


# speedup vs baseline: 1.3208x; 1.0973x over previous
import functools

import jax
import jax.numpy as jnp
from jax import lax
from jax.experimental import pallas as pl
from jax.experimental.pallas import tpu as pltpu
from jax.experimental.pallas import tpu_sc as plsc

N = 10000
E = 320000
C = 128
H = 4
HC = H * C

NC = 2
NS = 16
NW = NC * NS
EPT = 10048
B = 64
NBLK = EPT // B
E_PAD = NW * EPT
ND = H * EPT
NP_OUT = 10240

_mesh = plsc.VectorSubcoreMesh(
    core_axis_name="c", subcore_axis_name="s", num_cores=NC, num_subcores=NS)


def _pack_rows(y):
    bits = lax.bitcast_convert_type(y, jnp.int32)
    lo = lax.shift_right_logical(bits[:, :HC // 2] + 0x8000, 16)
    hi = jnp.bitwise_and(bits[:, HC // 2:] + 0x8000, jnp.int32(-65536))
    return jnp.bitwise_or(lo, hi)


def _mm_body(x_ref, wl_ref, wr_ref, bl_ref, br_ref, o_ref):
    t = pl.program_id(0)
    xv = x_ref[...]

    @pl.when(t == 0)
    def _l():
        o_ref[...] = _pack_rows(
            jnp.dot(xv, wl_ref[...], preferred_element_type=jnp.float32)
            + bl_ref[...])

    @pl.when(t == 1)
    def _r():
        o_ref[...] = _pack_rows(
            jnp.dot(xv, wr_ref[...], preferred_element_type=jnp.float32)
            + br_ref[...])


def _matmuls(x, W_l, W_r, b_l, b_r):
    blk = 1000
    nb = N // blk
    return pl.pallas_call(
        _mm_body,
        grid=(2, nb),
        in_specs=[
            pl.BlockSpec((blk, C), lambda t, i: (i, 0)),
            pl.BlockSpec((C, HC), lambda t, i: (0, 0)),
            pl.BlockSpec((C, HC), lambda t, i: (0, 0)),
            pl.BlockSpec((1, HC), lambda t, i: (0, 0)),
            pl.BlockSpec((1, HC), lambda t, i: (0, 0)),
        ],
        out_specs=pl.BlockSpec((blk, HC // 2), lambda t, i: (t * (N // 1000) + i, 0)),
        out_shape=jax.ShapeDtypeStruct((2 * N, HC // 2), jnp.int32),
    )(x, W_l, W_r, b_l.reshape(1, HC), b_r.reshape(1, HC))


def _p1_body(xc_hbm, pk_hbm, att_hbm,
             ex_hbm, den_hbm,
             idxb, xlbuf, lscr, exblk, att_v, den_v,
             sem1, semi, seme):
    c = lax.axis_index("c")
    s = lax.axis_index("s")
    wid = s * NC + c

    pltpu.sync_copy(att_hbm, att_v)
    attv = [plsc.bitcast(att_v[pl.ds(16 * p, 16)], jnp.bfloat16)
            for p in range(16)]
    zeros16 = jnp.zeros((16,), jnp.float32)

    @pl.loop(0, ND // 16)
    def _zero(i):
        den_v[pl.ds(i * 16, 16)] = zeros16

    gblk0 = wid * NBLK
    iota = lax.iota(jnp.int32, 16)
    onehot0 = (iota == 0).astype(jnp.float32)

    def start_gathers(bi, bg):
        pltpu.async_copy(xc_hbm.at[idxb.at[bi]], xlbuf.at[bg], sem1)

    def wait_gathers(bi, bg):
        pltpu.make_async_copy(xc_hbm.at[idxb.at[bi]], xlbuf.at[bg], sem1).wait()

    pltpu.sync_copy(pk_hbm.at[gblk0], idxb.at[0])
    start_gathers(0, 0)
    pltpu.async_copy(pk_hbm.at[gblk0 + 1], idxb.at[1], semi)

    @pl.loop(0, NBLK)
    def _blk(blk):
        par = blk & 1
        parn = 1 - par
        i_cur = lax.rem(blk, 3)
        i_next = lax.rem(blk + 1, 3)
        i_pref = lax.rem(blk + 2, 3)
        wait_gathers(i_cur, par)

        @pl.when(blk + 1 < NBLK)
        def _next():
            pltpu.make_async_copy(
                pk_hbm.at[gblk0 + blk + 1], idxb.at[i_next], semi).wait()
            start_gathers(i_next, parn)

        @pl.when(blk + 2 < NBLK)
        def _pref():
            pltpu.async_copy(pk_hbm.at[gblk0 + blk + 2], idxb.at[i_pref], semi)

        @pl.loop(0, B, unroll=4)
        def _edge(e):
            for h in range(H):
                accb = None
                for p4 in range(4):
                    p = h * 4 + p4
                    lw = plsc.bitcast(xlbuf[par, e, pl.ds(16 * p, 16)],
                                      jnp.bfloat16)
                    rw = plsc.bitcast(xlbuf[par, B + e, pl.ds(16 * p, 16)],
                                      jnp.bfloat16)
                    sv = lw + rw
                    lv = jnp.maximum(sv, jnp.bfloat16(0.2) * sv)
                    pr = lv * attv[p]
                    accb = pr if accb is None else accb + pr
                lo, hi = plsc.unpack(accb, format=plsc.PackFormat.INTERLEAVED)
                lscr[pl.ds(e * 64 + h * 16, 16)] = lo + hi

        @pl.when(blk >= 2)
        def _draine():
            pltpu.make_async_copy(
                exblk.at[pl.ds(0, B * 4)], ex_hbm.at[pl.ds(0, B * 4)], seme).wait()

        for g in range(B // 16):
            dstv = idxb[i_cur, pl.ds(B + g * 16, 16)] - N
            for h in range(H):
                tot = zeros16
                for j in range(16):
                    tot = tot + plsc.load_gather(
                        lscr, [iota * 64 + (g * 1024 + h * 16 + j)])
                exv = jnp.exp(tot)
                plsc.store_scatter(
                    exblk, [par * B * 4 + iota * 4 + (g * 64 + h)], exv)
                for j in range(16):
                    idx = dstv[j] * 4 + h
                    plsc.addupdate(den_v.at[pl.ds(idx, 16)], exv[j] * onehot0)

        base = (gblk0 + blk) * B
        pltpu.async_copy(exblk.at[pl.ds(par * B * 4, B * 4)],
                         ex_hbm.at[pl.ds(base * 4, B * 4)], seme)

    pltpu.make_async_copy(
        exblk.at[pl.ds(0, B * 4)], ex_hbm.at[pl.ds(0, B * 4)], seme).wait()
    pltpu.make_async_copy(
        exblk.at[pl.ds(0, B * 4)], ex_hbm.at[pl.ds(0, B * 4)], seme).wait()
    pltpu.sync_copy(den_v, den_hbm.at[wid])


@functools.partial(
    pl.kernel,
    out_type=(
        jax.ShapeDtypeStruct((E_PAD * 4,), jnp.float32),
        jax.ShapeDtypeStruct((NW, ND), jnp.float32),
    ),
    mesh=_mesh,
    scratch_types=[
        pltpu.VMEM((3, 2 * B), jnp.int32),
        pltpu.VMEM((2, 2 * B, HC // 2), jnp.int32),
        pltpu.VMEM((B * 64,), jnp.float32),
        pltpu.VMEM((2 * B * 4,), jnp.float32),
        pltpu.VMEM((HC // 2,), jnp.int32),
        pltpu.VMEM((ND,), jnp.float32),
        pltpu.SemaphoreType.DMA,
        pltpu.SemaphoreType.DMA,
        pltpu.SemaphoreType.DMA,
    ],
    compiler_params=pltpu.CompilerParams(needs_layout_passes=False),
)
def _phase1(*refs):
    _p1_body(*refs)


def _mid_body(den_ref, rden_ref):
    d = jnp.sum(den_ref[...], axis=0) + 1e-16
    rden_ref[...] = 1.0 / d


def _mid(den):
    return pl.pallas_call(
        _mid_body,
        out_shape=jax.ShapeDtypeStruct((ND // 128, 128), jnp.float32),
    )(den.reshape(NW, ND // 128, 128))


def _p2_body(ex_hbm, dst_hbm, rden_hbm, al_hbm, didx, exblk, rden_v):
    c = lax.axis_index("c")
    s = lax.axis_index("s")
    wid = s * NC + c
    base = wid * EPT

    pltpu.sync_copy(rden_hbm, rden_v)
    pltpu.sync_copy(dst_hbm.at[pl.ds(base, EPT)], didx)
    pltpu.sync_copy(ex_hbm.at[pl.ds(base * 4, EPT * 4)], exblk)
    iota = lax.iota(jnp.int32, 16)

    @pl.loop(0, EPT // 16)
    def _grp(g):
        dstv = didx[pl.ds(g * 16, 16)]
        dst4 = dstv * 4
        for h in range(H):
            exv = plsc.load_gather(exblk, [iota * 4 + (g * 64 + h)])
            rd = plsc.load_gather(rden_v, [dst4 + h])
            plsc.store_scatter(exblk, [iota * 4 + (g * 64 + h)], exv * rd)

    pltpu.sync_copy(exblk, al_hbm.at[pl.ds(base * 4, EPT * 4)])


@functools.partial(
    pl.kernel,
    out_type=jax.ShapeDtypeStruct((E_PAD * 4,), jnp.float32),
    mesh=_mesh,
    scratch_types=[
        pltpu.VMEM((EPT,), jnp.int32),
        pltpu.VMEM((EPT * 4,), jnp.float32),
        pltpu.VMEM((ND,), jnp.float32),
    ],
    compiler_params=pltpu.CompilerParams(needs_layout_passes=False),
)
def _phase2(*refs):
    _p2_body(*refs)


def _p3_body(xl_hbm, pk_hbm, al_hbm,
             outp_hbm,
             idxb, xlbuf, albuf, vbuf, out_sh, sem1, sema, semi):
    c = lax.axis_index("c")
    s = lax.axis_index("s")
    wid = s * NC + c

    zeros16 = jnp.zeros((16,), jnp.float32)

    @pl.loop(0, B)
    def _zv(e):
        for j8 in range(8):
            vbuf[e, pl.ds(16 * j8, 16)] = zeros16

    @pl.loop(0, NP_OUT // (NS * B))
    def _zo(t):
        pltpu.sync_copy(vbuf, out_sh.at[pl.ds(s * (NP_OUT // NS) + t * B, B)])

    plsc.subcore_barrier()

    gblk0 = wid * NBLK

    def start_fetch(bi, bg, gblk):
        pltpu.async_copy(xl_hbm.at[idxb.at[bi, 0]], xlbuf.at[bg], sem1)
        pltpu.async_copy(al_hbm.at[pl.ds(gblk * (B * 4), B * 4)],
                         albuf.at[bg, pl.ds(0, B * 4)], sema)

    def wait_fetch(bi, bg):
        pltpu.make_async_copy(xl_hbm.at[idxb.at[bi, 0]], xlbuf.at[bg], sem1).wait()
        pltpu.make_async_copy(al_hbm.at[pl.ds(0, B * 4)],
                              albuf.at[bg, pl.ds(0, B * 4)], sema).wait()

    pltpu.sync_copy(pk_hbm.at[gblk0], idxb.at[0])
    start_fetch(0, 0, gblk0)
    pltpu.async_copy(pk_hbm.at[gblk0 + 1], idxb.at[1], semi)

    @pl.loop(0, NBLK)
    def _blk(blk):
        par = blk & 1
        parn = 1 - par
        i_cur = lax.rem(blk, 3)
        i_next = lax.rem(blk + 1, 3)
        i_pref = lax.rem(blk + 2, 3)
        wait_fetch(i_cur, par)

        @pl.when(blk + 1 < NBLK)
        def _next():
            pltpu.make_async_copy(
                pk_hbm.at[gblk0 + blk + 1], idxb.at[i_next], semi).wait()
            start_fetch(i_next, parn, gblk0 + blk + 1)

        @pl.when(blk + 2 < NBLK)
        def _pref():
            pltpu.async_copy(pk_hbm.at[gblk0 + blk + 2], idxb.at[i_pref], semi)

        @pl.loop(0, B, unroll=4)
        def _edge(e):
            av = albuf[par, pl.ds(e * 4, 16)]
            aa = [av[0], av[1], av[2], av[3]]
            for p in range(4):
                acclo = jnp.zeros((16,), jnp.float32)
                acchi = jnp.zeros((16,), jnp.float32)
                for h in range(H):
                    xh = plsc.bitcast(
                        xlbuf[par, e, pl.ds(h * 64 + p * 16, 16)], jnp.bfloat16)
                    lo, hi = plsc.unpack(xh, format=plsc.PackFormat.INTERLEAVED)
                    acclo = acclo + aa[h] * lo
                    acchi = acchi + aa[h] * hi
                vbuf[e, pl.ds(p * 32, 16)] = acclo
                vbuf[e, pl.ds(p * 32 + 16, 16)] = acchi

        pltpu.sync_copy(vbuf, out_sh.at[idxb.at[i_cur, 1]], add=True)

    plsc.subcore_barrier()
    rows = NP_OUT // NS
    pltpu.sync_copy(out_sh.at[pl.ds(s * rows, rows)],
                    outp_hbm.at[c, pl.ds(s * rows, rows)])


@functools.partial(
    pl.kernel,
    out_type=jax.ShapeDtypeStruct((NC, NP_OUT, C), jnp.float32),
    mesh=_mesh,
    scratch_types=[
        pltpu.VMEM((3, 2, B), jnp.int32),
        pltpu.VMEM((2, B, HC // 2), jnp.int32),
        pltpu.VMEM((2, B * 4 + 16), jnp.float32),
        pltpu.VMEM((B, C), jnp.float32),
        pltpu.VMEM_SHARED((NP_OUT, C), jnp.float32),
        pltpu.SemaphoreType.DMA,
        pltpu.SemaphoreType.DMA,
        pltpu.SemaphoreType.DMA,
    ],
    compiler_params=pltpu.CompilerParams(needs_layout_passes=False),
)
def _phase3(*refs):
    _p3_body(*refs)


def _fin_body(p_ref, x_ref, b_ref, gw_ref, gb_ref, gms_ref, o_ref):
    p = p_ref[0, :N, :] + p_ref[1, :N, :]
    out0 = p * (1.0 / H) + b_ref[...]
    mean = jnp.mean(out0, axis=0, keepdims=True)
    outc = out0 - gms_ref[...] * mean
    var = jnp.mean(outc * outc, axis=0, keepdims=True)
    y = outc * lax.rsqrt(var + 1e-5) * gw_ref[...] + gb_ref[...]
    y = jnp.where(y > 0, y, jnp.exp(y) - 1.0)
    o_ref[...] = y + x_ref[...]


def _final(outp, x, bias, gn_weight, gn_bias, gn_mean_scale):
    return pl.pallas_call(
        _fin_body,
        out_shape=jax.ShapeDtypeStruct((N, C), jnp.float32),
    )(outp, x, bias.reshape(1, C), gn_weight.reshape(1, C),
      gn_bias.reshape(1, C), gn_mean_scale.reshape(1, C))


def kernel(x, edge_index, W_l, b_l, W_r, b_r, att, bias, gn_weight,
           gn_bias, gn_mean_scale):
    ei = edge_index.astype(jnp.int32)
    pad = E_PAD - E
    src = jnp.concatenate([ei[0], jnp.zeros((pad,), jnp.int32)])
    dst = jnp.concatenate([ei[1], jnp.full((pad,), N, jnp.int32)])
    pk1 = jnp.concatenate(
        [src.reshape(-1, B), dst.reshape(-1, B) + N], axis=1)
    pk3 = jnp.stack([src.reshape(-1, B), dst.reshape(-1, B)], axis=1)

    perm = jnp.asarray(
        [32 * (w // 16) + (w % 16) for w in range(HC // 2)]
        + [32 * (w // 16) + 16 + (w % 16) for w in range(HC // 2)],
        dtype=jnp.int32)
    perm_att = jnp.asarray(
        [32 * g + off for g in range(HC // 32) for k in range(16)
         for off in (k, 16 + k)], dtype=jnp.int32)
    Wlp = jnp.take(W_l, perm, axis=1)
    Wrp = jnp.take(W_r, perm, axis=1)
    blp = jnp.take(b_l, perm)
    brp = jnp.take(b_r, perm)
    attp = lax.bitcast_convert_type(
        jnp.take(att.reshape(HC), perm_att).astype(jnp.bfloat16)
        .reshape(HC // 2, 2), jnp.int32)

    xc32 = _matmuls(x, Wlp, Wrp, blp, brp)
    ex, den = _phase1(xc32, pk1, attp)
    rden = _mid(den)
    al = _phase2(ex, dst, rden.reshape(ND))
    outp = _phase3(xc32, pk3, al)
    return _final(outp, x, bias, gn_weight, gn_bias, gn_mean_scale)
